# Initial kernel scaffold; baseline (speedup 1.0000x reference)
import jax
import jax.numpy as jnp
from jax.experimental import pallas as pl


def kernel(x, edge_index, batch, W1, b1, a1, W2, b2, a2):
    raise NotImplementedError("write your pallas kernel here")


# speedup vs baseline: 16.3573x; 16.3573x over previous
import functools

import jax
import jax.numpy as jnp
from jax import lax
from jax.experimental import pallas as pl
from jax.experimental.pallas import tpu as pltpu
from jax.experimental.pallas import tpu_sc as plsc

N = 10000
E = 320000
D = 128
G = 128
DH = D // 2

NC = 2
NS = 16
NW = NC * NS
CHUNK = 80
EPW = E // NW
NCHUNK = EPW // CHUNK
NP = 10240
RPT = NP // NS
ZROWS = 128

MBLK = 1000
NBLK = N // MBLK

_MESH = plsc.VectorSubcoreMesh(core_axis_name="c", subcore_axis_name="s")
_SC_PARAMS = pltpu.CompilerParams(use_tc_tiling_on_sc=False)


@functools.partial(
    pl.kernel,
    out_type=jax.ShapeDtypeStruct((NC, NP, 16), jnp.float32),
    mesh=_MESH,
    scratch_types=[
        pltpu.VMEM((NCHUNK, CHUNK), jnp.int32),
        pltpu.VMEM((CHUNK, 16), jnp.float32),
        pltpu.VMEM((ZROWS, 16), jnp.float32),
        pltpu.VMEM_SHARED((NP, 16), jnp.float32),
        pltpu.SemaphoreType.DMA,
    ],
    compiler_params=_SC_PARAMS,
)
def _deg(dst_hbm, out_hbm, dst_v, ones_v, z_v, acc, sem):
    c = lax.axis_index("c")
    s = lax.axis_index("s")
    wid = c * NS + s

    @pl.loop(0, CHUNK)
    def _(r):
        ones_v[r, :] = jnp.ones((16,), jnp.float32)

    @pl.loop(0, ZROWS)
    def _(r):
        z_v[r, :] = jnp.zeros((16,), jnp.float32)

    for k in range(RPT // ZROWS):
        pltpu.sync_copy(z_v, acc.at[pl.ds(s * RPT + k * ZROWS, ZROWS)])
    plsc.subcore_barrier()

    pltpu.sync_copy(dst_hbm.at[wid], dst_v)

    @pl.loop(0, NCHUNK)
    def _(j):
        pltpu.sync_copy(ones_v, acc.at[dst_v.at[j]], add=True)

    plsc.subcore_barrier()
    pltpu.sync_copy(acc.at[pl.ds(s * RPT, RPT)],
                    out_hbm.at[c, pl.ds(s * RPT, RPT)])


@functools.partial(
    pl.kernel,
    out_type=jax.ShapeDtypeStruct((NC, 2, NP, DH), jnp.float32),
    mesh=_MESH,
    scratch_types=[
        pltpu.VMEM((NCHUNK, CHUNK), jnp.int32),
        pltpu.VMEM((NCHUNK, CHUNK), jnp.int32),
        pltpu.VMEM((CHUNK, DH), jnp.float32),
        pltpu.VMEM((CHUNK, DH), jnp.float32),
        pltpu.VMEM((ZROWS, DH), jnp.float32),
        pltpu.VMEM_SHARED((NP, DH), jnp.float32),
        pltpu.SemaphoreType.DMA,
        pltpu.SemaphoreType.DMA,
    ],
    compiler_params=_SC_PARAMS,
)
def _edge_pass(y_hbm, src_hbm, dst_hbm, out_hbm,
               src_v, dst_v, buf0, buf1, z_v, acc, sem0, sem1):
    c = lax.axis_index("c")
    s = lax.axis_index("s")
    wid = c * NS + s

    @pl.loop(0, ZROWS)
    def _(r):
        for cb in range(DH // 16):
            z_v[r, pl.ds(cb * 16, 16)] = jnp.zeros((16,), jnp.float32)

    pltpu.sync_copy(src_hbm.at[wid], src_v)
    pltpu.sync_copy(dst_hbm.at[wid], dst_v)

    for h in range(2):
        tab = y_hbm.at[h]
        for k in range(RPT // ZROWS):
            pltpu.sync_copy(z_v, acc.at[pl.ds(s * RPT + k * ZROWS, ZROWS)])
        plsc.subcore_barrier()

        pltpu.async_copy(tab.at[src_v.at[0]], buf0, sem0)

        @pl.loop(0, (NCHUNK - 1) // 2)
        def _(i):
            j0 = 2 * i
            pltpu.make_async_copy(tab.at[src_v.at[j0]], buf0, sem0).wait()
            pltpu.async_copy(tab.at[src_v.at[j0 + 1]], buf1, sem1)
            pltpu.sync_copy(buf0, acc.at[dst_v.at[j0]], add=True)
            pltpu.make_async_copy(tab.at[src_v.at[j0 + 1]], buf1, sem1).wait()
            pltpu.async_copy(tab.at[src_v.at[j0 + 2]], buf0, sem0)
            pltpu.sync_copy(buf1, acc.at[dst_v.at[j0 + 1]], add=True)

        pltpu.make_async_copy(tab.at[src_v.at[NCHUNK - 1]], buf0, sem0).wait()
        pltpu.sync_copy(buf0, acc.at[dst_v.at[NCHUNK - 1]], add=True)

        plsc.subcore_barrier()
        pltpu.sync_copy(acc.at[pl.ds(s * RPT, RPT)],
                        out_hbm.at[c, h, pl.ds(s * RPT, RPT)])


def _dinv_blk(deg_ref):
    d = deg_ref[...]
    return lax.rsqrt(1.0 + d[0, :, 0:1] + d[1, :, 0:1])


def _split_cols(v):
    return jnp.stack([v[:, :DH], v[:, DH:]], axis=0)


def _merge(o_ref, y_ref):
    o = o_ref[...]
    y = y_ref[...]
    t = o[0] + o[1] + y
    return jnp.concatenate([t[0], t[1]], axis=1)


def _mm_scale_body(x_ref, w_ref, deg_ref, y_ref):
    xw = jnp.dot(x_ref[...], w_ref[...],
                 preferred_element_type=jnp.float32,
                 precision=lax.Precision.HIGHEST)
    y_ref[...] = _split_cols(xw * _dinv_blk(deg_ref))


def _mm_scale(x, w, degp):
    return pl.pallas_call(
        _mm_scale_body,
        grid=(NBLK,),
        in_specs=[
            pl.BlockSpec((MBLK, D), lambda i: (i, 0)),
            pl.BlockSpec((D, D), lambda i: (0, 0)),
            pl.BlockSpec((NC, MBLK, 16), lambda i: (0, i, 0)),
        ],
        out_specs=pl.BlockSpec((2, MBLK, DH), lambda i: (0, i, 0)),
        out_shape=jax.ShapeDtypeStruct((2, N, DH), jnp.float32),
    )(x, w, degp)


def _pool_update(i, g_ref, bt_ref, z):
    @pl.when(i == 0)
    def _():
        g_ref[...] = jnp.zeros_like(g_ref)

    bb = bt_ref[0, 0, :]
    gi = lax.broadcasted_iota(jnp.int32, (G, MBLK), 0)
    m = (gi == bb[None, :]).astype(jnp.float32)
    g_ref[...] += jnp.dot(m, z, preferred_element_type=jnp.float32,
                          precision=lax.Precision.HIGHEST)


def _combine_mm_pool_body(o_ref, y_ref, deg_ref, b_ref, a_ref, w_ref, bt_ref,
                          z_ref, y2_ref, g_ref):
    i = pl.program_id(0)
    dinv = _dinv_blk(deg_ref)
    u = _merge(o_ref, y_ref) * dinv + b_ref[...]
    z = jnp.where(u >= 0, u, a_ref[...] * u)
    z_ref[...] = z
    y2 = jnp.dot(z, w_ref[...],
                 preferred_element_type=jnp.float32,
                 precision=lax.Precision.HIGHEST) * dinv
    y2_ref[...] = _split_cols(y2)
    _pool_update(i, g_ref, bt_ref, z)


def _combine_mm_pool(o, y, degp, b, a, w, batch3):
    return pl.pallas_call(
        _combine_mm_pool_body,
        grid=(NBLK,),
        in_specs=[
            pl.BlockSpec((NC, 2, MBLK, DH), lambda i: (0, 0, i, 0)),
            pl.BlockSpec((2, MBLK, DH), lambda i: (0, i, 0)),
            pl.BlockSpec((NC, MBLK, 16), lambda i: (0, i, 0)),
            pl.BlockSpec((1, D), lambda i: (0, 0)),
            pl.BlockSpec((1, D), lambda i: (0, 0)),
            pl.BlockSpec((D, D), lambda i: (0, 0)),
            pl.BlockSpec((1, 1, MBLK), lambda i: (i, 0, 0)),
        ],
        out_specs=[
            pl.BlockSpec((MBLK, D), lambda i: (i, 0)),
            pl.BlockSpec((2, MBLK, DH), lambda i: (0, i, 0)),
            pl.BlockSpec((G, D), lambda i: (0, 0)),
        ],
        out_shape=[
            jax.ShapeDtypeStruct((N, D), jnp.float32),
            jax.ShapeDtypeStruct((2, N, DH), jnp.float32),
            jax.ShapeDtypeStruct((G, D), jnp.float32),
        ],
    )(o, y, degp, b, a, w, batch3)


def _combine_pool_body(o_ref, y_ref, deg_ref, b_ref, a_ref, bt_ref,
                       z_ref, g_ref):
    i = pl.program_id(0)
    dinv = _dinv_blk(deg_ref)
    u = _merge(o_ref, y_ref) * dinv + b_ref[...]
    z = jnp.where(u >= 0, u, a_ref[...] * u)
    z_ref[...] = z
    _pool_update(i, g_ref, bt_ref, z)


def _combine_pool(o, y, degp, b, a, batch3):
    return pl.pallas_call(
        _combine_pool_body,
        grid=(NBLK,),
        in_specs=[
            pl.BlockSpec((NC, 2, MBLK, DH), lambda i: (0, 0, i, 0)),
            pl.BlockSpec((2, MBLK, DH), lambda i: (0, i, 0)),
            pl.BlockSpec((NC, MBLK, 16), lambda i: (0, i, 0)),
            pl.BlockSpec((1, D), lambda i: (0, 0)),
            pl.BlockSpec((1, D), lambda i: (0, 0)),
            pl.BlockSpec((1, 1, MBLK), lambda i: (i, 0, 0)),
        ],
        out_specs=[
            pl.BlockSpec((MBLK, D), lambda i: (i, 0)),
            pl.BlockSpec((G, D), lambda i: (0, 0)),
        ],
        out_shape=[
            jax.ShapeDtypeStruct((N, D), jnp.float32),
            jax.ShapeDtypeStruct((G, D), jnp.float32),
        ],
    )(o, y, degp, b, a, batch3)


@jax.jit
def kernel(x, edge_index, batch, W1, b1, a1, W2, b2, a2):
    src3 = edge_index[0].reshape(NW, NCHUNK, CHUNK)
    dst3 = edge_index[1].reshape(NW, NCHUNK, CHUNK)
    batch3 = batch.reshape(NBLK, 1, MBLK)
    b1r = b1.reshape(1, D)
    a1r = a1.reshape(1, D)
    b2r = b2.reshape(1, D)
    a2r = a2.reshape(1, D)

    degp = _deg(dst3)
    y1 = _mm_scale(x, W1, degp)
    o1 = _edge_pass(y1, src3, dst3)
    z1, y2, g1 = _combine_mm_pool(o1, y1, degp, b1r, a1r, W2, batch3)
    o2 = _edge_pass(y2, src3, dst3)
    z2, g2 = _combine_pool(o2, y2, degp, b2r, a2r, batch3)

    g = jnp.concatenate([g1, g2], axis=1)
    return (z2, g)


# speedup vs baseline: 22.8389x; 1.3962x over previous
import functools

import jax
import jax.numpy as jnp
from jax import lax
from jax.experimental import pallas as pl
from jax.experimental.pallas import tpu as pltpu
from jax.experimental.pallas import tpu_sc as plsc

N = 10000
E = 320000
D = 128
G = 128
DH = D // 2

NC = 2
NS = 16
NW = NC * NS
CHUNK = 80
EPW = E // NW
NCHUNK = EPW // CHUNK
NP = 10240
RPT = NP // NS
ZROWS = 128

MBLK = 1000
NBLK = N // MBLK

_MESH = plsc.VectorSubcoreMesh(core_axis_name="c", subcore_axis_name="s")
_SC_PARAMS = pltpu.CompilerParams(use_tc_tiling_on_sc=False)


@functools.partial(
    pl.kernel,
    out_type=jax.ShapeDtypeStruct((NC, NP, 16), jnp.float32),
    mesh=_MESH,
    scratch_types=[
        pltpu.VMEM((NCHUNK, CHUNK), jnp.int32),
        pltpu.VMEM((CHUNK, 16), jnp.float32),
        pltpu.VMEM((ZROWS, 16), jnp.float32),
        pltpu.VMEM_SHARED((NP, 16), jnp.float32),
        pltpu.SemaphoreType.DMA,
    ],
    compiler_params=_SC_PARAMS,
)
def _deg(dst_hbm, out_hbm, dst_v, ones_v, z_v, acc, sem):
    c = lax.axis_index("c")
    s = lax.axis_index("s")
    wid = c * NS + s

    @pl.loop(0, CHUNK)
    def _(r):
        ones_v[r, :] = jnp.ones((16,), jnp.float32)

    @pl.loop(0, ZROWS)
    def _(r):
        z_v[r, :] = jnp.zeros((16,), jnp.float32)

    for k in range(RPT // ZROWS):
        pltpu.sync_copy(z_v, acc.at[pl.ds(s * RPT + k * ZROWS, ZROWS)])
    plsc.subcore_barrier()

    pltpu.sync_copy(dst_hbm.at[wid], dst_v)

    @pl.loop(0, NCHUNK)
    def _(j):
        pltpu.async_copy(ones_v, acc.at[dst_v.at[j]], sem, add=True)

    @pl.loop(0, NCHUNK)
    def _(j):
        pltpu.make_async_copy(ones_v, acc.at[dst_v.at[j]], sem).wait()

    plsc.subcore_barrier()
    pltpu.sync_copy(acc.at[pl.ds(s * RPT, RPT)],
                    out_hbm.at[c, pl.ds(s * RPT, RPT)])


NBUF = 5
GAHEAD = 2
assert NCHUNK % NBUF == 0 and NCHUNK >= 3 * NBUF


@functools.partial(
    pl.kernel,
    out_type=jax.ShapeDtypeStruct((NC, 2, NP, DH), jnp.float32),
    mesh=_MESH,
    scratch_types=[
        pltpu.VMEM((NCHUNK, CHUNK), jnp.int32),
        pltpu.VMEM((NCHUNK, CHUNK), jnp.int32),
    ] + [pltpu.VMEM((CHUNK, DH), jnp.float32) for _ in range(NBUF)]
      + [pltpu.VMEM((ZROWS, DH), jnp.float32),
         pltpu.VMEM_SHARED((NP, DH), jnp.float32)]
      + [pltpu.SemaphoreType.DMA for _ in range(2 * NBUF)],
    compiler_params=_SC_PARAMS,
)
def _edge_pass(y_hbm, src_hbm, dst_hbm, out_hbm,
               src_v, dst_v, b0, b1, b2, b3, b4, z_v, acc,
               g0, g1, g2, g3, g4, s0, s1, s2, s3, s4):
    c = lax.axis_index("c")
    s = lax.axis_index("s")
    wid = c * NS + s
    bufs = (b0, b1, b2, b3, b4)
    gsems = (g0, g1, g2, g3, g4)
    ssems = (s0, s1, s2, s3, s4)

    @pl.loop(0, ZROWS)
    def _(r):
        for cb in range(DH // 16):
            z_v[r, pl.ds(cb * 16, 16)] = jnp.zeros((16,), jnp.float32)

    pltpu.sync_copy(src_hbm.at[wid], src_v)
    pltpu.sync_copy(dst_hbm.at[wid], dst_v)

    for h in range(2):
        tab = y_hbm.at[h]

        def gath(j, b):
            pltpu.async_copy(tab.at[src_v.at[j]], bufs[b], gsems[b])

        def gwait(j, b):
            pltpu.make_async_copy(tab.at[src_v.at[j]], bufs[b],
                                  gsems[b]).wait()

        def scat(j, b):
            pltpu.async_copy(bufs[b], acc.at[dst_v.at[j]], ssems[b],
                             add=True)

        def swait(j, b):
            pltpu.make_async_copy(bufs[b], acc.at[dst_v.at[j]],
                                  ssems[b]).wait()

        def step(j, b, do_swait, do_gath):
            gwait(j, b)
            scat(j, b)
            bn = (b + GAHEAD) % NBUF
            if do_swait:
                swait(j - (NBUF - GAHEAD), bn)
            if do_gath:
                gath(j + GAHEAD, bn)

        for k in range(RPT // ZROWS):
            pltpu.sync_copy(z_v, acc.at[pl.ds(s * RPT + k * ZROWS, ZROWS)])
        plsc.subcore_barrier()

        for j in range(GAHEAD):
            gath(j, j)
        for j in range(NBUF):
            step(j, j, j >= NBUF - GAHEAD, True)

        @pl.loop(1, NCHUNK // NBUF - 1)
        def _(grp):
            j0 = grp * NBUF
            for k in range(NBUF):
                step(j0 + k, k, True, True)

        for j in range(NCHUNK - NBUF, NCHUNK):
            b = j % NBUF
            step(j, b, j < NCHUNK - GAHEAD, j < NCHUNK - GAHEAD)
        for j in range(NCHUNK - NBUF, NCHUNK):
            swait(j, j % NBUF)

        plsc.subcore_barrier()
        pltpu.sync_copy(acc.at[pl.ds(s * RPT, RPT)],
                        out_hbm.at[c, h, pl.ds(s * RPT, RPT)])


def _dinv_blk(deg_ref):
    d = deg_ref[...]
    return lax.rsqrt(1.0 + d[0, :, 0:1] + d[1, :, 0:1])


def _split_cols(v):
    return jnp.stack([v[:, :DH], v[:, DH:]], axis=0)


def _merge(o_ref, y_ref):
    o = o_ref[...]
    y = y_ref[...]
    t = o[0] + o[1] + y
    return jnp.concatenate([t[0], t[1]], axis=1)


def _mm_scale_body(x_ref, w_ref, deg_ref, y_ref):
    xw = jnp.dot(x_ref[...], w_ref[...],
                 preferred_element_type=jnp.float32,
                 precision=lax.Precision.HIGHEST)
    y_ref[...] = _split_cols(xw * _dinv_blk(deg_ref))


def _mm_scale(x, w, degp):
    return pl.pallas_call(
        _mm_scale_body,
        grid=(NBLK,),
        in_specs=[
            pl.BlockSpec((MBLK, D), lambda i: (i, 0)),
            pl.BlockSpec((D, D), lambda i: (0, 0)),
            pl.BlockSpec((NC, MBLK, 16), lambda i: (0, i, 0)),
        ],
        out_specs=pl.BlockSpec((2, MBLK, DH), lambda i: (0, i, 0)),
        out_shape=jax.ShapeDtypeStruct((2, N, DH), jnp.float32),
    )(x, w, degp)


def _pool_update(i, g_ref, bt_ref, z):
    @pl.when(i == 0)
    def _():
        g_ref[...] = jnp.zeros_like(g_ref)

    bb = bt_ref[0, 0, :]
    gi = lax.broadcasted_iota(jnp.int32, (G, MBLK), 0)
    m = (gi == bb[None, :]).astype(jnp.float32)
    g_ref[...] += jnp.dot(m, z, preferred_element_type=jnp.float32,
                          precision=lax.Precision.HIGHEST)


def _combine_mm_pool_body(o_ref, y_ref, deg_ref, b_ref, a_ref, w_ref, bt_ref,
                          z_ref, y2_ref, g_ref):
    i = pl.program_id(0)
    dinv = _dinv_blk(deg_ref)
    u = _merge(o_ref, y_ref) * dinv + b_ref[...]
    z = jnp.where(u >= 0, u, a_ref[...] * u)
    z_ref[...] = z
    y2 = jnp.dot(z, w_ref[...],
                 preferred_element_type=jnp.float32,
                 precision=lax.Precision.HIGHEST) * dinv
    y2_ref[...] = _split_cols(y2)
    _pool_update(i, g_ref, bt_ref, z)


def _combine_mm_pool(o, y, degp, b, a, w, batch3):
    return pl.pallas_call(
        _combine_mm_pool_body,
        grid=(NBLK,),
        in_specs=[
            pl.BlockSpec((NC, 2, MBLK, DH), lambda i: (0, 0, i, 0)),
            pl.BlockSpec((2, MBLK, DH), lambda i: (0, i, 0)),
            pl.BlockSpec((NC, MBLK, 16), lambda i: (0, i, 0)),
            pl.BlockSpec((1, D), lambda i: (0, 0)),
            pl.BlockSpec((1, D), lambda i: (0, 0)),
            pl.BlockSpec((D, D), lambda i: (0, 0)),
            pl.BlockSpec((1, 1, MBLK), lambda i: (i, 0, 0)),
        ],
        out_specs=[
            pl.BlockSpec((MBLK, D), lambda i: (i, 0)),
            pl.BlockSpec((2, MBLK, DH), lambda i: (0, i, 0)),
            pl.BlockSpec((G, D), lambda i: (0, 0)),
        ],
        out_shape=[
            jax.ShapeDtypeStruct((N, D), jnp.float32),
            jax.ShapeDtypeStruct((2, N, DH), jnp.float32),
            jax.ShapeDtypeStruct((G, D), jnp.float32),
        ],
    )(o, y, degp, b, a, w, batch3)


def _combine_pool_body(o_ref, y_ref, deg_ref, b_ref, a_ref, bt_ref,
                       z_ref, g_ref):
    i = pl.program_id(0)
    dinv = _dinv_blk(deg_ref)
    u = _merge(o_ref, y_ref) * dinv + b_ref[...]
    z = jnp.where(u >= 0, u, a_ref[...] * u)
    z_ref[...] = z
    _pool_update(i, g_ref, bt_ref, z)


def _combine_pool(o, y, degp, b, a, batch3):
    return pl.pallas_call(
        _combine_pool_body,
        grid=(NBLK,),
        in_specs=[
            pl.BlockSpec((NC, 2, MBLK, DH), lambda i: (0, 0, i, 0)),
            pl.BlockSpec((2, MBLK, DH), lambda i: (0, i, 0)),
            pl.BlockSpec((NC, MBLK, 16), lambda i: (0, i, 0)),
            pl.BlockSpec((1, D), lambda i: (0, 0)),
            pl.BlockSpec((1, D), lambda i: (0, 0)),
            pl.BlockSpec((1, 1, MBLK), lambda i: (i, 0, 0)),
        ],
        out_specs=[
            pl.BlockSpec((MBLK, D), lambda i: (i, 0)),
            pl.BlockSpec((G, D), lambda i: (0, 0)),
        ],
        out_shape=[
            jax.ShapeDtypeStruct((N, D), jnp.float32),
            jax.ShapeDtypeStruct((G, D), jnp.float32),
        ],
    )(o, y, degp, b, a, batch3)


@jax.jit
def kernel(x, edge_index, batch, W1, b1, a1, W2, b2, a2):
    src3 = edge_index[0].reshape(NW, NCHUNK, CHUNK)
    dst3 = edge_index[1].reshape(NW, NCHUNK, CHUNK)
    batch3 = batch.reshape(NBLK, 1, MBLK)
    b1r = b1.reshape(1, D)
    a1r = a1.reshape(1, D)
    b2r = b2.reshape(1, D)
    a2r = a2.reshape(1, D)

    degp = _deg(dst3)
    y1 = _mm_scale(x, W1, degp)
    o1 = _edge_pass(y1, src3, dst3)
    z1, y2, g1 = _combine_mm_pool(o1, y1, degp, b1r, a1r, W2, batch3)
    o2 = _edge_pass(y2, src3, dst3)
    z2, g2 = _combine_pool(o2, y2, degp, b2r, a2r, batch3)

    g = jnp.concatenate([g1, g2], axis=1)
    return (z2, g)
